# Initial kernel scaffold; baseline (speedup 1.0000x reference)
import jax
import jax.numpy as jnp
from jax.experimental import pallas as pl


def kernel(x, edge_index, W1, b1, W2, b2):
    raise NotImplementedError("write your pallas kernel here")


# speedup vs baseline: 39.7669x; 39.7669x over previous
import functools

import jax
import jax.numpy as jnp
from jax import lax
from jax.experimental import pallas as pl
from jax.experimental.pallas import tpu as pltpu
from jax.experimental.pallas import tpu_sc as plsc

N_NODES = 10000
N_EDGES = 320000
D_IN = 128
D_HID = 64

NC = 2
NS = 16
NW = NC * NS
E_PER_W = N_EDGES // NW
CHUNK = 80
NCHUNK = E_PER_W // CHUNK
N_PAD = 10240
ROWS_PER_TILE = N_PAD // NS
LANES = 16
D_PAD = 128

_mesh = plsc.VectorSubcoreMesh(core_axis_name="c", subcore_axis_name="s")


@functools.partial(
    pl.kernel,
    out_type=jax.ShapeDtypeStruct((NW * N_NODES,), jnp.float32),
    mesh=_mesh,
    scratch_types=[
        pltpu.VMEM((NCHUNK, CHUNK), jnp.int32),
        pltpu.VMEM((N_NODES,), jnp.float32),
    ],
    compiler_params=pltpu.CompilerParams(needs_layout_passes=False),
)
def _deg_kernel(dst_hbm, out_hbm, dst_v, acc_l):
    c = lax.axis_index("c")
    s = lax.axis_index("s")
    wid = c * NS + s
    pltpu.sync_copy(dst_hbm.at[wid], dst_v)

    zeros16 = jnp.zeros((LANES,), jnp.float32)

    def zbody(i, _):
        acc_l[pl.ds(i * LANES, LANES)] = zeros16
        return 0

    lax.fori_loop(0, N_NODES // LANES, zbody, 0)

    ones16 = jnp.ones((LANES,), jnp.float32)

    def ebody(r, _):
        for k in range(CHUNK // LANES):
            idx = dst_v[r, pl.ds(k * LANES, LANES)]
            plsc.addupdate_scatter(acc_l, [idx], ones16)
        return 0

    lax.fori_loop(0, NCHUNK, ebody, 0)
    pltpu.sync_copy(acc_l, out_hbm.at[pl.ds(wid * N_NODES, N_NODES)])


@functools.partial(
    pl.kernel,
    out_type=jax.ShapeDtypeStruct((NC, N_PAD, D_HID), jnp.float32),
    mesh=_mesh,
    scratch_types=[
        pltpu.VMEM((NCHUNK, CHUNK), jnp.int32),
        pltpu.VMEM((NCHUNK, CHUNK), jnp.int32),
        pltpu.VMEM((CHUNK, D_HID), jnp.float32),
        pltpu.VMEM((ROWS_PER_TILE, D_HID), jnp.float32),
        pltpu.VMEM_SHARED((N_PAD, D_HID), jnp.float32),
        pltpu.SemaphoreType.DMA,
    ],
    compiler_params=pltpu.CompilerParams(needs_layout_passes=False,
                                         use_tc_tiling_on_sc=False),
)
def _agg1_kernel(g_hbm, src_hbm, dst_hbm, zero_hbm, out_hbm,
                 src_v, dst_v, rows_v, stage_v, acc_sh, sem):
    c = lax.axis_index("c")
    s = lax.axis_index("s")
    wid = c * NS + s
    pltpu.sync_copy(src_hbm.at[wid], src_v)
    pltpu.sync_copy(dst_hbm.at[wid], dst_v)

    pltpu.sync_copy(zero_hbm, stage_v)
    pltpu.sync_copy(stage_v, acc_sh.at[pl.ds(s * ROWS_PER_TILE, ROWS_PER_TILE)])
    plsc.subcore_barrier()

    def ebody(j, _):
        pltpu.async_copy(g_hbm.at[src_v.at[j]], rows_v, sem).wait()
        pltpu.sync_copy(rows_v, acc_sh.at[dst_v.at[j]], add=True)
        return 0

    lax.fori_loop(0, NCHUNK, ebody, 0)
    plsc.subcore_barrier()

    pltpu.sync_copy(acc_sh.at[pl.ds(s * ROWS_PER_TILE, ROWS_PER_TILE)], stage_v)
    pltpu.sync_copy(stage_v,
                    out_hbm.at[c, pl.ds(s * ROWS_PER_TILE, ROWS_PER_TILE)])


@functools.partial(
    pl.kernel,
    out_type=jax.ShapeDtypeStruct((NW * N_NODES,), jnp.float32),
    mesh=_mesh,
    scratch_types=[
        pltpu.VMEM((NCHUNK, CHUNK), jnp.int32),
        pltpu.VMEM((NCHUNK, CHUNK), jnp.int32),
        pltpu.VMEM((N_NODES,), jnp.float32),
        pltpu.VMEM((N_NODES,), jnp.float32),
    ],
    compiler_params=pltpu.CompilerParams(needs_layout_passes=False),
)
def _agg2_kernel(zd_hbm, src_hbm, dst_hbm, out_hbm, src_v, dst_v, zd_v, acc_l):
    c = lax.axis_index("c")
    s = lax.axis_index("s")
    wid = c * NS + s
    pltpu.sync_copy(zd_hbm, zd_v)
    pltpu.sync_copy(src_hbm.at[wid], src_v)
    pltpu.sync_copy(dst_hbm.at[wid], dst_v)

    zeros16 = jnp.zeros((LANES,), jnp.float32)

    def zbody(i, _):
        acc_l[pl.ds(i * LANES, LANES)] = zeros16
        return 0

    lax.fori_loop(0, N_NODES // LANES, zbody, 0)

    def ebody(r, _):
        for k in range(CHUNK // LANES):
            si = src_v[r, pl.ds(k * LANES, LANES)]
            di = dst_v[r, pl.ds(k * LANES, LANES)]
            vals = plsc.load_gather(zd_v, [si])
            plsc.addupdate_scatter(acc_l, [di], vals)
        return 0

    lax.fori_loop(0, NCHUNK, ebody, 0)
    pltpu.sync_copy(acc_l, out_hbm.at[pl.ds(wid * N_NODES, N_NODES)])


def _dense1_body(x_ref, w1_ref, degp_ref, g_ref, dinv_ref):
    ones = jnp.ones((NW, 1), jnp.float32)
    deg = lax.dot_general(degp_ref[...], ones, (((0,), (0,)), ((), ())),
                          preferred_element_type=jnp.float32) + 1.0
    dinv = lax.rsqrt(deg)
    h = jnp.dot(x_ref[...], w1_ref[...], preferred_element_type=jnp.float32)
    g_ref[...] = h * dinv
    dinv_ref[...] = dinv


_dense1 = pl.pallas_call(
    _dense1_body,
    out_shape=[
        jax.ShapeDtypeStruct((N_NODES, D_HID), jnp.float32),
        jax.ShapeDtypeStruct((N_NODES, 1), jnp.float32),
    ],
)


def _dense2_body(acc_ref, g_ref, dinv_ref, b1_ref, w2_ref, zd_ref):
    t = (acc_ref[0, :N_NODES, :D_HID] + acc_ref[1, :N_NODES, :D_HID]
         + g_ref[:, :D_HID])
    out1 = jnp.maximum(t * dinv_ref[...] + b1_ref[...][None, :], 0.0)
    z = jnp.dot(out1, w2_ref[...], preferred_element_type=jnp.float32)
    zd_ref[...] = z * dinv_ref[...]


_dense2 = pl.pallas_call(
    _dense2_body,
    out_shape=jax.ShapeDtypeStruct((N_NODES, 1), jnp.float32),
)


def _dense3_body(accp_ref, zd_ref, dinv_ref, b2_ref, out_ref):
    ones = jnp.ones((NW, 1), jnp.float32)
    acc2 = lax.dot_general(accp_ref[...], ones, (((0,), (0,)), ((), ())),
                           preferred_element_type=jnp.float32)
    v = (acc2 + zd_ref[...]) * dinv_ref[...] + b2_ref[...]
    out_ref[...] = jax.nn.sigmoid(v)


_dense3 = pl.pallas_call(
    _dense3_body,
    out_shape=jax.ShapeDtypeStruct((N_NODES, 1), jnp.float32),
)


def kernel(x, edge_index, W1, b1, W2, b2):
    src = edge_index[0].astype(jnp.int32).reshape(NW, NCHUNK, CHUNK)
    dst = edge_index[1].astype(jnp.int32).reshape(NW, NCHUNK, CHUNK)
    zeros = jnp.zeros((ROWS_PER_TILE, D_HID), jnp.float32)

    degp = _deg_kernel(dst).reshape(NW, N_NODES)
    g, dinv = _dense1(x, W1, degp)
    acc = _agg1_kernel(g, src, dst, zeros)
    zd = _dense2(acc, g, dinv, b1, W2)
    accp = _agg2_kernel(zd.reshape(N_NODES), src, dst).reshape(NW, N_NODES)
    out = _dense3(accp, zd, dinv, b2)
    return out
